# baseline (speedup 1.0000x reference)
import functools

import jax
import jax.numpy as jnp
import numpy as np
from jax import lax
from jax.experimental import pallas as pl
from jax.experimental.pallas import tpu as pltpu
from jax.experimental.pallas import tpu_sc as plsc

_N = 10000
_E = 320000
_D = 128
_NUM_SAMPLES = 5

_NC = 2
_NS = 16
_NW = _NC * _NS
_EPT = _E // _NW
_C = 80
_NCHUNK = _EPT // _C
_NROWCHUNK = _N // _C
_RCPT = -(-_NROWCHUNK // _NS)

_PERM = np.empty((_D,), np.int32)
for _q in range(_D // 32):
    for _i in range(16):
        _PERM[32 * _q + 2 * _i] = 32 * _q + _i
        _PERM[32 * _q + 2 * _i + 1] = 32 * _q + 16 + _i


def _mm_body(x_ref, w_ref, o_ref):
    r = jnp.dot(x_ref[...], w_ref[...], preferred_element_type=jnp.float32)
    o_ref[...] = r.astype(jnp.bfloat16)


def _matmul(x, W):
    bm = 1000
    return pl.pallas_call(
        _mm_body,
        out_shape=jax.ShapeDtypeStruct((_N, _D), jnp.bfloat16),
        grid=(_N // bm,),
        in_specs=[
            pl.BlockSpec((bm, _D), lambda i: (i, 0)),
            pl.BlockSpec((_D, _D), lambda i: (0, 0)),
        ],
        out_specs=pl.BlockSpec((bm, _D), lambda i: (i, 0)),
    )(x, W)


_MESH = plsc.VectorSubcoreMesh(core_axis_name="c", subcore_axis_name="s",
                               num_cores=_NC, num_subcores=_NS)


@functools.partial(
    pl.kernel,
    out_type=jax.ShapeDtypeStruct((_NC, _N, _D), jnp.float32),
    mesh=_MESH,
    compiler_params=pltpu.CompilerParams(needs_layout_passes=False,
                                         use_tc_tiling_on_sc=False),
    scratch_types=(
        [pltpu.VMEM((_C,), jnp.int32)] * 6
        + [pltpu.VMEM((_C,), jnp.int32)] * 6
        + [pltpu.VMEM((_C,), jnp.float32)] * 6
        + [pltpu.VMEM((_C, _D // 2), jnp.int32)] * 3
        + [pltpu.VMEM((_C, _D), jnp.float32)] * 3
        + [pltpu.VMEM_SHARED((_N, _D), jnp.float32)]
        + [pltpu.SemaphoreType.DMA] * 12
    ),
)
def _spmm(rows_hbm, cols_hbm, vals_hbm, h_hbm, out_hbm,
          r0, r1, r2, r3, r4, r5, c0, c1, c2, c3, c4, c5,
          v0, v1, v2, v3, v4, v5,
          gb0, gb1, gb2, sb0, sb1, sb2, acc_sh,
          is0, is1, is2, is3, is4, is5,
          gs0, gs1, gs2, ss0, ss1, ss2):
    cid = lax.axis_index("c")
    sid = lax.axis_index("s")
    wid = sid * _NC + cid
    row_v = (r0, r1, r2, r3, r4, r5)
    col_v = (c0, c1, c2, c3, c4, c5)
    val_v = (v0, v1, v2, v3, v4, v5)
    gb = (gb0, gb1, gb2)
    sb = (sb0, sb1, sb2)
    isem = (is0, is1, is2, is3, is4, is5)
    gsem = (gs0, gs1, gs2)
    ssem = (ss0, ss1, ss2)

    def _idx_start(k, s):
        base = wid * _EPT + k * _C
        pltpu.async_copy(rows_hbm.at[pl.ds(base, _C)], row_v[s], isem[s])
        pltpu.async_copy(cols_hbm.at[pl.ds(base, _C)], col_v[s], isem[s])
        pltpu.async_copy(vals_hbm.at[pl.ds(base, _C)], val_v[s], isem[s])

    def _idx_wait(k, s):
        base = wid * _EPT + k * _C
        pltpu.make_async_copy(rows_hbm.at[pl.ds(base, _C)], row_v[s],
                              isem[s]).wait()
        pltpu.make_async_copy(cols_hbm.at[pl.ds(base, _C)], col_v[s],
                              isem[s]).wait()
        pltpu.make_async_copy(vals_hbm.at[pl.ds(base, _C)], val_v[s],
                              isem[s]).wait()

    _idx_start(0, 0)

    zero16 = jnp.zeros((16,), jnp.float32)

    def _zero_buf(r, carry):
        for j in range(_D // 16):
            sb0[r, pl.ds(j * 16, 16)] = zero16
        return carry

    lax.fori_loop(0, _C, _zero_buf, 0)
    for i in range(_RCPT):
        rc = sid * _RCPT + i

        @pl.when(rc < _NROWCHUNK)
        def _():
            pltpu.sync_copy(sb0, acc_sh.at[pl.ds(rc * _C, _C)])

    plsc.subcore_barrier()

    def _scale(b, s):
        gbuf = gb[b]
        sbuf = sb[b]
        vbuf = val_v[s]

        def _grp(gi, c2):
            vals16 = vbuf[pl.ds(gi * 16, 16)]
            for j in range(16):
                e = gi * 16 + j
                vs = jnp.full((16,), vals16[j], jnp.float32)
                for p in range(_D // 32):
                    w = plsc.bitcast(gbuf[e, pl.ds(p * 16, 16)],
                                     jnp.bfloat16)
                    lo, hi = plsc.unpack(w, format=plsc.PackFormat.INTERLEAVED)
                    sbuf[e, pl.ds(p * 32, 16)] = lo * vs
                    sbuf[e, pl.ds(p * 32 + 16, 16)] = hi * vs
            return c2

        lax.fori_loop(0, _C // 16, _grp, 0)

    def _scat_wait(b, s):
        pltpu.make_async_copy(sb[b], acc_sh.at[row_v[s]], ssem[b]).wait()

    def _maybe(cond, fn):
        if isinstance(cond, bool):
            if cond:
                fn()
        else:
            pl.when(cond)(fn)

    def _step(k, b, s):
        b1 = (b + 1) % 3
        s1 = (s + 1) % 6
        s3 = (s + 3) % 6

        _maybe(k >= 3, lambda: _scat_wait(b, s3))

        _maybe(k + 3 < _NCHUNK, lambda: _idx_start(k + 3, s3))

        def _launch_next():
            _idx_wait(k + 1, s1)
            pltpu.async_copy(h_hbm.at[col_v[s1]], gb[b1], gsem[b1])

        _maybe(k + 1 < _NCHUNK, _launch_next)

        pltpu.make_async_copy(h_hbm.at[col_v[s]], gb[b], gsem[b]).wait()
        _scale(b, s)
        pltpu.async_copy(sb[b], acc_sh.at[row_v[s]], ssem[b], add=True)

    _idx_start(1, 1)
    _idx_wait(0, 0)
    pltpu.async_copy(h_hbm.at[col_v[0]], gb0, gsem[0])
    _idx_start(2, 2)

    def _six(t, carry):
        for j in range(6):
            _step(6 * t + j, j % 3, j)
        return carry

    lax.fori_loop(0, _NCHUNK // 6, _six, 0)
    for k in range(_NCHUNK - _NCHUNK % 6, _NCHUNK):
        _step(k, k % 3, k % 6)
    _scat_wait(2, 2)
    _scat_wait(0, 3)
    _scat_wait(1, 4)
    plsc.subcore_barrier()

    for i in range(_RCPT):
        rc = sid * _RCPT + i

        @pl.when(rc < _NROWCHUNK)
        def _():
            pltpu.sync_copy(acc_sh.at[pl.ds(rc * _C, _C)],
                            out_hbm.at[cid, pl.ds(rc * _C, _C)])


def _ep_body(p_ref, b_ref, g_ref, be_ref, o_ref):
    agg = p_ref[0] + p_ref[1]
    h2 = agg + b_ref[...]
    h2 = jnp.where(h2 > 0, h2, jnp.exp(jnp.minimum(h2, 0.0)) - 1.0)
    mu = jnp.mean(h2, axis=-1, keepdims=True)
    d = h2 - mu
    var = jnp.mean(d * d, axis=-1, keepdims=True)
    hn = d * lax.rsqrt(var + 1e-5) * g_ref[...] + be_ref[...]
    o_ref[...] = jnp.broadcast_to(hn[:, None, :],
                                  (hn.shape[0], _NUM_SAMPLES, _D))


def _epilogue(partials, b, ln_gamma, ln_beta):
    bm = 400
    return pl.pallas_call(
        _ep_body,
        out_shape=jax.ShapeDtypeStruct((_N, _NUM_SAMPLES, _D), jnp.float32),
        grid=(_N // bm,),
        in_specs=[
            pl.BlockSpec((_NC, bm, _D), lambda i: (0, i, 0)),
            pl.BlockSpec((1, _D), lambda i: (0, 0)),
            pl.BlockSpec((_D,), lambda i: (0,)),
            pl.BlockSpec((_D,), lambda i: (0,)),
        ],
        out_specs=pl.BlockSpec((bm, _NUM_SAMPLES, _D), lambda i: (i, 0, 0)),
    )(partials, b, ln_gamma, ln_beta)


def kernel(adj_indices, adj_values, x, W, b, ln_gamma, ln_beta):
    h_bf = _matmul(x, jnp.take(W, jnp.asarray(_PERM), axis=1))
    h_packed = jax.lax.bitcast_convert_type(
        h_bf.reshape(_N, _D // 2, 2), jnp.int32)
    partials = _spmm(adj_indices[0], adj_indices[1], adj_values, h_packed)
    return _epilogue(partials, b, ln_gamma, ln_beta)


# speedup vs baseline: 1.8344x; 1.8344x over previous
import functools

import jax
import jax.numpy as jnp
from jax import lax
from jax.experimental import pallas as pl
from jax.experimental.pallas import tpu as pltpu
from jax.experimental.pallas import tpu_sc as plsc

_N = 10000
_E = 320000
_D = 128
_NUM_SAMPLES = 5

_NC = 2
_NS = 16
_NW = _NC * _NS
_EPT = _E // _NW
_C = 80
_NCHUNK = _EPT // _C
_NROWCHUNK = _N // _C
_RCPT = -(-_NROWCHUNK // _NS)


def _mm_body(x_ref, w_ref, o_ref):
    o_ref[...] = jnp.dot(x_ref[...], w_ref[...],
                         preferred_element_type=jnp.float32)


def _matmul(x, W):
    bm = 1000
    return pl.pallas_call(
        _mm_body,
        out_shape=jax.ShapeDtypeStruct((_N, _D), jnp.float32),
        grid=(_N // bm,),
        in_specs=[
            pl.BlockSpec((bm, _D), lambda i: (i, 0)),
            pl.BlockSpec((_D, _D), lambda i: (0, 0)),
        ],
        out_specs=pl.BlockSpec((bm, _D), lambda i: (i, 0)),
    )(x, W)


_MESH = plsc.VectorSubcoreMesh(core_axis_name="c", subcore_axis_name="s",
                               num_cores=_NC, num_subcores=_NS)


@functools.partial(
    pl.kernel,
    out_type=jax.ShapeDtypeStruct((_NC, _N, _D), jnp.float32),
    mesh=_MESH,
    scratch_types=(
        [pltpu.VMEM((_C,), jnp.int32)] * 4
        + [pltpu.VMEM((_C,), jnp.int32)] * 4
        + [pltpu.VMEM((_C,), jnp.float32)] * 4
        + [pltpu.VMEM((_C, _D), jnp.float32)] * 4
        + [pltpu.VMEM_SHARED((_N, _D), jnp.float32)]
        + [pltpu.SemaphoreType.DMA] * 12
    ),
)
def _spmm(rows_hbm, cols_hbm, vals_hbm, h_hbm, out_hbm,
          r0, r1, r2, r3, c0, c1, c2, c3, v0, v1, v2, v3,
          g0, g1, g2, g3, acc_sh,
          is0, is1, is2, is3, gs0, gs1, gs2, gs3, ss0, ss1, ss2, ss3):
    cid = lax.axis_index("c")
    sid = lax.axis_index("s")
    wid = sid * _NC + cid
    row_v = (r0, r1, r2, r3)
    col_v = (c0, c1, c2, c3)
    val_v = (v0, v1, v2, v3)
    g = (g0, g1, g2, g3)
    isem = (is0, is1, is2, is3)
    gsem = (gs0, gs1, gs2, gs3)
    ssem = (ss0, ss1, ss2, ss3)

    def _idx_start(k, b):
        base = wid * _EPT + k * _C
        pltpu.async_copy(rows_hbm.at[pl.ds(base, _C)], row_v[b], isem[b])
        pltpu.async_copy(cols_hbm.at[pl.ds(base, _C)], col_v[b], isem[b])
        pltpu.async_copy(vals_hbm.at[pl.ds(base, _C)], val_v[b], isem[b])

    def _idx_wait(k, b):
        base = wid * _EPT + k * _C
        pltpu.make_async_copy(rows_hbm.at[pl.ds(base, _C)], row_v[b],
                              isem[b]).wait()
        pltpu.make_async_copy(cols_hbm.at[pl.ds(base, _C)], col_v[b],
                              isem[b]).wait()
        pltpu.make_async_copy(vals_hbm.at[pl.ds(base, _C)], val_v[b],
                              isem[b]).wait()

    _idx_start(0, 0)

    zero16 = jnp.zeros((16,), jnp.float32)

    def _zero_buf(r, carry):
        for j in range(_D // 16):
            g0[r, pl.ds(j * 16, 16)] = zero16
        return carry

    lax.fori_loop(0, _C, _zero_buf, 0)
    for i in range(_RCPT):
        rc = sid * _RCPT + i

        @pl.when(rc < _NROWCHUNK)
        def _():
            pltpu.sync_copy(g0, acc_sh.at[pl.ds(rc * _C, _C)])

    plsc.subcore_barrier()

    def _scale(buf, vbuf):
        def _grp(gi, c2):
            vals16 = vbuf[pl.ds(gi * 16, 16)]
            for j in range(16):
                e = gi * 16 + j
                vs = jnp.full((16,), vals16[j], jnp.float32)
                for q in range(_D // 16):
                    buf[e, pl.ds(q * 16, 16)] = buf[e, pl.ds(q * 16, 16)] * vs
            return c2

        lax.fori_loop(0, _C // 16, _grp, 0)

    def _scat_wait(b):
        pltpu.make_async_copy(g[b], acc_sh.at[row_v[b]], ssem[b]).wait()

    def _step(k, b):
        b1 = (b + 1) % 4
        b2 = (b + 2) % 4

        @pl.when(k >= 2)
        def _():
            _scat_wait(b2)

        @pl.when(k + 1 < _NCHUNK)
        def _():
            _idx_wait(k + 1, b1)
            pltpu.async_copy(h_hbm.at[col_v[b1]], g[b1], gsem[b1])

        @pl.when(k + 2 < _NCHUNK)
        def _():
            _idx_start(k + 2, b2)

        pltpu.make_async_copy(h_hbm.at[col_v[b]], g[b], gsem[b]).wait()
        _scale(g[b], val_v[b])
        pltpu.async_copy(g[b], acc_sh.at[row_v[b]], ssem[b], add=True)

    _idx_wait(0, 0)
    pltpu.async_copy(h_hbm.at[col_v[0]], g0, gsem[0])
    _idx_start(1, 1)

    def _quad(t, carry):
        for b in range(4):
            _step(4 * t + b, b)
        return carry

    lax.fori_loop(0, _NCHUNK // 4, _quad, 0)
    _step(_NCHUNK - 1, (_NCHUNK - 1) % 4)
    _scat_wait((_NCHUNK - 2) % 4)
    _scat_wait((_NCHUNK - 1) % 4)
    plsc.subcore_barrier()

    for i in range(_RCPT):
        rc = sid * _RCPT + i

        @pl.when(rc < _NROWCHUNK)
        def _():
            pltpu.sync_copy(acc_sh.at[pl.ds(rc * _C, _C)],
                            out_hbm.at[cid, pl.ds(rc * _C, _C)])


def _ep_body(p_ref, b_ref, g_ref, be_ref, o_ref):
    agg = p_ref[0] + p_ref[1]
    h2 = agg + b_ref[...]
    h2 = jnp.where(h2 > 0, h2, jnp.exp(jnp.minimum(h2, 0.0)) - 1.0)
    mu = jnp.mean(h2, axis=-1, keepdims=True)
    d = h2 - mu
    var = jnp.mean(d * d, axis=-1, keepdims=True)
    hn = d * lax.rsqrt(var + 1e-5) * g_ref[...] + be_ref[...]
    o_ref[...] = jnp.broadcast_to(hn[:, None, :],
                                  (hn.shape[0], _NUM_SAMPLES, _D))


def _epilogue(partials, b, ln_gamma, ln_beta):
    bm = 400
    return pl.pallas_call(
        _ep_body,
        out_shape=jax.ShapeDtypeStruct((_N, _NUM_SAMPLES, _D), jnp.float32),
        grid=(_N // bm,),
        in_specs=[
            pl.BlockSpec((_NC, bm, _D), lambda i: (0, i, 0)),
            pl.BlockSpec((1, _D), lambda i: (0, 0)),
            pl.BlockSpec((_D,), lambda i: (0,)),
            pl.BlockSpec((_D,), lambda i: (0,)),
        ],
        out_specs=pl.BlockSpec((bm, _NUM_SAMPLES, _D), lambda i: (i, 0, 0)),
    )(partials, b, ln_gamma, ln_beta)


def kernel(adj_indices, adj_values, x, W, b, ln_gamma, ln_beta):
    h = _matmul(x, W)
    partials = _spmm(adj_indices[0], adj_indices[1], adj_values, h)
    return _epilogue(partials, b, ln_gamma, ln_beta)
